# Initial kernel scaffold; baseline (speedup 1.0000x reference)
import jax
import jax.numpy as jnp
from jax.experimental import pallas as pl


def kernel(idxes, edges, weights, pos):
    raise NotImplementedError("write your pallas kernel here")


# speedup vs baseline: 37.9454x; 37.9454x over previous
import functools

import jax
import jax.numpy as jnp
from jax import lax
from jax.experimental import pallas as pl
from jax.experimental.pallas import tpu as pltpu
from jax.experimental.pallas import tpu_sc as plsc


N_NODES = 10000
NDIM = 2
BATCH = 1024
DEG = 32
K2 = 1.0 / N_NODES
INV_K = float(N_NODES) ** 0.5

NC, NS, L = 2, 16, 16
NW = NC * NS
BPW = BATCH // NW
EPW = BPW * DEG

N_PAD = 10240
BT = 256
NT = 1024


def _sc_gather_body(posx_hbm, posy_hbm, idx_hbm, edg_hbm,
                    pbx_hbm, pby_hbm, ex_hbm, ey_hbm,
                    posx_v, posy_v, idx_v, edg_v,
                    pbx_v, pby_v, ex_v, ey_v):
    wid = lax.axis_index("s") * NC + lax.axis_index("c")
    base = wid * BPW
    ebase = wid * EPW
    pltpu.sync_copy(posx_hbm, posx_v)
    pltpu.sync_copy(posy_hbm, posy_v)
    pltpu.sync_copy(idx_hbm.at[pl.ds(base, BPW)], idx_v)
    pltpu.sync_copy(edg_hbm.at[pl.ds(ebase, EPW)], edg_v)

    for g in range(BPW // L):
        iv = idx_v[pl.ds(g * L, L)]
        pbx_v[pl.ds(g * L, L)] = plsc.load_gather(posx_v, [iv])
        pby_v[pl.ds(g * L, L)] = plsc.load_gather(posy_v, [iv])

    def egroup(g, c):
        ev = edg_v[pl.ds(g * L, L)]
        ex_v[pl.ds(g * L, L)] = plsc.load_gather(posx_v, [ev])
        ey_v[pl.ds(g * L, L)] = plsc.load_gather(posy_v, [ev])
        return c
    lax.fori_loop(0, EPW // L, egroup, 0)

    pltpu.sync_copy(pbx_v, pbx_hbm.at[pl.ds(base, BPW)])
    pltpu.sync_copy(pby_v, pby_hbm.at[pl.ds(base, BPW)])
    pltpu.sync_copy(ex_v, ex_hbm.at[pl.ds(ebase, EPW)])
    pltpu.sync_copy(ey_v, ey_hbm.at[pl.ds(ebase, EPW)])


_sc_gather = pl.kernel(
    _sc_gather_body,
    out_type=[
        jax.ShapeDtypeStruct((BATCH,), jnp.float32),
        jax.ShapeDtypeStruct((BATCH,), jnp.float32),
        jax.ShapeDtypeStruct((BATCH * DEG,), jnp.float32),
        jax.ShapeDtypeStruct((BATCH * DEG,), jnp.float32),
    ],
    mesh=plsc.VectorSubcoreMesh(core_axis_name="c", subcore_axis_name="s"),
    compiler_params=pltpu.CompilerParams(needs_layout_passes=False),
    scratch_types=[
        pltpu.VMEM((N_NODES,), jnp.float32),
        pltpu.VMEM((N_NODES,), jnp.float32),
        pltpu.VMEM((BPW,), jnp.int32),
        pltpu.VMEM((EPW,), jnp.int32),
        pltpu.VMEM((BPW,), jnp.float32),
        pltpu.VMEM((BPW,), jnp.float32),
        pltpu.VMEM((EPW,), jnp.float32),
        pltpu.VMEM((EPW,), jnp.float32),
    ],
)


def _tc_body(pbx_ref, pby_ref, posx_ref, posy_ref,
             ex_ref, ey_ref, w_ref, dux_ref, duy_ref):
    j = pl.program_id(1)
    px = pbx_ref[...]
    py = pby_ref[...]

    dx = px - posx_ref[...]
    dy = py - posy_ref[...]
    d2 = jnp.maximum(dx * dx + dy * dy, K2 * K2 * 1.0e4)
    nid = lax.broadcasted_iota(jnp.int32, (BT, NT), 1) + j * NT
    inv = jnp.where(nid < N_NODES, K2 / d2, 0.0)
    rx = (dx * inv).sum(axis=1, keepdims=True)
    ry = (dy * inv).sum(axis=1, keepdims=True)

    @pl.when(j == 0)
    def _():
        exv = ex_ref[...]
        eyv = ey_ref[...]
        dxa = px - exv
        dya = py - eyv
        da = jnp.maximum(jnp.sqrt(dxa * dxa + dya * dya), 0.01)
        coef = da * w_ref[...] * INV_K
        dux_ref[...] = rx - (dxa * coef).sum(axis=1, keepdims=True)
        duy_ref[...] = ry - (dya * coef).sum(axis=1, keepdims=True)

    @pl.when(j != 0)
    def _():
        dux_ref[...] += rx
        duy_ref[...] += ry


_tc_compute = pl.pallas_call(
    _tc_body,
    grid=(BATCH // BT, N_PAD // NT),
    in_specs=[
        pl.BlockSpec((BT, 1), lambda i, j: (i, 0)),
        pl.BlockSpec((BT, 1), lambda i, j: (i, 0)),
        pl.BlockSpec((1, NT), lambda i, j: (0, j)),
        pl.BlockSpec((1, NT), lambda i, j: (0, j)),
        pl.BlockSpec((BT, DEG), lambda i, j: (i, 0)),
        pl.BlockSpec((BT, DEG), lambda i, j: (i, 0)),
        pl.BlockSpec((BT, DEG), lambda i, j: (i, 0)),
    ],
    out_specs=[
        pl.BlockSpec((BT, 1), lambda i, j: (i, 0)),
        pl.BlockSpec((BT, 1), lambda i, j: (i, 0)),
    ],
    out_shape=[
        jax.ShapeDtypeStruct((BATCH, 1), jnp.float32),
        jax.ShapeDtypeStruct((BATCH, 1), jnp.float32),
    ],
    compiler_params=pltpu.CompilerParams(
        dimension_semantics=("parallel", "arbitrary"),
    ),
)


def _sc_scatter_body(idx_hbm, dux_hbm, duy_hbm, zero_hbm, out_hbm,
                     idx_v, dx_v, dy_v, disp_v):
    is_w0 = jnp.logical_and(lax.axis_index("c") == 0, lax.axis_index("s") == 0)

    @pl.when(is_w0)
    def _():
        pltpu.sync_copy(zero_hbm, disp_v)
        pltpu.sync_copy(idx_hbm, idx_v)
        pltpu.sync_copy(dux_hbm, dx_v)
        pltpu.sync_copy(duy_hbm, dy_v)

        iota16 = lax.iota(jnp.int32, L)

        def p2(g, c):
            ivec = idx_v[pl.ds(g * L, L)]
            xv = dx_v[pl.ds(g * L, L)]
            yv = dy_v[pl.ds(g * L, L)]
            ix = ivec * 2
            iy = ix + 1
            for lane in range(L):
                m = iota16 == lane
                plsc.store_scatter(disp_v, [ix], xv, mask=m)
                plsc.store_scatter(disp_v, [iy], yv, mask=m)
            return c
        lax.fori_loop(0, BATCH // L, p2, 0)

        pltpu.sync_copy(disp_v, out_hbm)


_sc_scatter = pl.kernel(
    _sc_scatter_body,
    out_type=jax.ShapeDtypeStruct((N_NODES * NDIM,), jnp.float32),
    mesh=plsc.VectorSubcoreMesh(core_axis_name="c", subcore_axis_name="s"),
    compiler_params=pltpu.CompilerParams(needs_layout_passes=False),
    scratch_types=[
        pltpu.VMEM((BATCH,), jnp.int32),
        pltpu.VMEM((BATCH,), jnp.float32),
        pltpu.VMEM((BATCH,), jnp.float32),
        pltpu.VMEM((N_NODES * NDIM,), jnp.float32),
    ],
)


def kernel(idxes, edges, weights, pos):
    idx32 = idxes.astype(jnp.int32)
    edg32 = edges.astype(jnp.int32).reshape(-1)
    posx = pos[:, 0]
    posy = pos[:, 1]

    pbx, pby, ex, ey = _sc_gather(posx, posy, idx32, edg32)

    posx_p = jnp.pad(posx, (0, N_PAD - N_NODES)).reshape(1, N_PAD)
    posy_p = jnp.pad(posy, (0, N_PAD - N_NODES)).reshape(1, N_PAD)
    dux, duy = _tc_compute(
        pbx.reshape(BATCH, 1), pby.reshape(BATCH, 1),
        posx_p, posy_p,
        ex.reshape(BATCH, DEG), ey.reshape(BATCH, DEG),
        weights,
    )

    disp = _sc_scatter(
        idx32, dux.reshape(-1), duy.reshape(-1),
        jnp.zeros((N_NODES * NDIM,), jnp.float32),
    )
    return disp.reshape(N_NODES, NDIM)
